# Initial kernel scaffold; baseline (speedup 1.0000x reference)
import jax
import jax.numpy as jnp
from jax.experimental import pallas as pl


def kernel(x, table):
    raise NotImplementedError("write your pallas kernel here")


# speedup vs baseline: 2.9489x; 2.9489x over previous
import functools

import jax
import jax.numpy as jnp
from jax import lax
from jax.experimental import pallas as pl
from jax.experimental.pallas import tpu as pltpu
from jax.experimental.pallas import tpu_sc as plsc

D = 32
B = 16384 * 50

NC, NS = 2, 16
NW = NC * NS
BPW = B // NW

ICHUNK = 128
KI = 8
BLOCK = ICHUNK * KI
NBLK = BPW // BLOCK

_mesh = plsc.VectorSubcoreMesh(core_axis_name="c", subcore_axis_name="s")


@functools.partial(
    pl.kernel,
    out_type=jax.ShapeDtypeStruct((B, D), jnp.float32),
    mesh=_mesh,
    compiler_params=pltpu.CompilerParams(use_tc_tiling_on_sc=False),
    scratch_types=[
        pltpu.VMEM((BLOCK,), jnp.int32),
        pltpu.VMEM((BLOCK, D), jnp.float32),
        pltpu.SemaphoreType.DMA,
    ],
)
def _gather_kernel(x_hbm, table_hbm, out_hbm, idx_v, rows_v, sem):
    wid = lax.axis_index("s") * NC + lax.axis_index("c")
    base = wid * BPW

    def body(g, carry):
        off = base + g * BLOCK
        pltpu.sync_copy(x_hbm.at[pl.ds(off, BLOCK)], idx_v)
        copies = []
        for j in range(KI):
            copies.append(
                pltpu.async_copy(
                    table_hbm.at[idx_v.at[pl.ds(j * ICHUNK, ICHUNK)]],
                    rows_v.at[pl.ds(j * ICHUNK, ICHUNK)],
                    sem,
                )
            )
        for c in copies:
            c.wait()
        pltpu.sync_copy(rows_v, out_hbm.at[pl.ds(off, BLOCK)])
        return carry

    lax.fori_loop(0, NBLK, body, 0)


def kernel(x, table):
    out = _gather_kernel(x.reshape(B), table)
    return out.reshape(x.shape + (D,))


# speedup vs baseline: 3.0044x; 1.0188x over previous
import functools

import jax
import jax.numpy as jnp
from jax import lax
from jax.experimental import pallas as pl
from jax.experimental.pallas import tpu as pltpu
from jax.experimental.pallas import tpu_sc as plsc

D = 32
B = 16384 * 50

NC, NS = 2, 16
NW = NC * NS
BPW = B // NW

ICHUNK = 128
KI = 10
BLOCK = ICHUNK * KI
NBLK = BPW // BLOCK
NPAIR = NBLK // 2 - 1

_mesh = plsc.VectorSubcoreMesh(core_axis_name="c", subcore_axis_name="s")


@functools.partial(
    pl.kernel,
    out_type=jax.ShapeDtypeStruct((B, D), jnp.float32),
    mesh=_mesh,
    compiler_params=pltpu.CompilerParams(use_tc_tiling_on_sc=False),
    scratch_types=[
        pltpu.VMEM((BPW,), jnp.int32),
        pltpu.VMEM((BLOCK, D), jnp.float32),
        pltpu.VMEM((BLOCK, D), jnp.float32),
        pltpu.SemaphoreType.DMA,
        pltpu.SemaphoreType.DMA,
        pltpu.SemaphoreType.DMA,
        pltpu.SemaphoreType.DMA,
    ],
)
def _gather_kernel(x_hbm, table_hbm, out_hbm, idx_v, rows0, rows1, sg0, sg1,
                   so0, so1):
    wid = lax.axis_index("s") * NC + lax.axis_index("c")
    base = wid * BPW
    rows = (rows0, rows1)
    sg = (sg0, sg1)
    so = (so0, so1)

    def fire_gathers(g, slot):
        for j in range(KI):
            pltpu.async_copy(
                table_hbm.at[idx_v.at[pl.ds(g * BLOCK + j * ICHUNK, ICHUNK)]],
                rows[slot].at[pl.ds(j * ICHUNK, ICHUNK)],
                sg[slot],
            )

    def wait_gathers(g, slot):
        pltpu.make_async_copy(
            out_hbm.at[pl.ds(base + g * BLOCK, BLOCK)], rows[slot], sg[slot]
        ).wait()

    def fire_scatter(g, slot):
        pltpu.async_copy(
            rows[slot], out_hbm.at[pl.ds(base + g * BLOCK, BLOCK)], so[slot]
        )

    def wait_scatter(g, slot):
        pltpu.make_async_copy(
            rows[slot], out_hbm.at[pl.ds(base + g * BLOCK, BLOCK)], so[slot]
        ).wait()

    pltpu.sync_copy(x_hbm.at[pl.ds(base, BPW)], idx_v)
    fire_gathers(0, 0)
    fire_gathers(1, 1)

    def body(p, carry):
        g = p * 2
        wait_gathers(g, 0)
        fire_scatter(g, 0)
        wait_gathers(g + 1, 1)
        wait_scatter(g, 0)
        fire_gathers(g + 2, 0)
        fire_scatter(g + 1, 1)
        wait_scatter(g + 1, 1)
        fire_gathers(g + 3, 1)
        return carry

    lax.fori_loop(0, NPAIR, body, 0)

    g = NBLK - 2
    wait_gathers(g, 0)
    fire_scatter(g, 0)
    wait_gathers(g + 1, 1)
    fire_scatter(g + 1, 1)
    wait_scatter(g, 0)
    wait_scatter(g + 1, 1)


def kernel(x, table):
    out = _gather_kernel(x.reshape(B), table)
    return out.reshape(x.shape + (D,))


# speedup vs baseline: 4.8555x; 1.6161x over previous
import functools

import jax
import jax.numpy as jnp
from jax import lax
from jax.experimental import pallas as pl
from jax.experimental.pallas import tpu as pltpu
from jax.experimental.pallas import tpu_sc as plsc

D = 32
SEQ = 50
BT = 128
NBT = 16384 // BT
B = 16384 * SEQ

NC, NS = 2, 16
NW = NC * NS
NITEM = SEQ * NBT
IPW = NITEM // NW
NPAIR = IPW // 2

OUTROW = 4 * BT * 8 * BT

_mesh = plsc.VectorSubcoreMesh(core_axis_name="c", subcore_axis_name="s")


@functools.partial(
    pl.kernel,
    out_type=jax.ShapeDtypeStruct((SEQ, OUTROW), jnp.float32),
    mesh=_mesh,
    compiler_params=pltpu.CompilerParams(
        use_tc_tiling_on_sc=False, needs_layout_passes=False
    ),
    scratch_types=[
        pltpu.VMEM((IPW * BT,), jnp.int32),
        pltpu.VMEM((BT, D), jnp.float32),
        pltpu.VMEM((BT, D), jnp.float32),
        pltpu.VMEM((4, 8 * BT), jnp.float32),
        pltpu.VMEM((4, 8 * BT), jnp.float32),
        pltpu.SemaphoreType.DMA,
        pltpu.SemaphoreType.DMA,
        pltpu.SemaphoreType.DMA,
        pltpu.SemaphoreType.DMA,
    ],
)
def _gather_kernel(x_hbm, table_hbm, out_hbm, idx_v, rows0, rows1, t0, t1,
                   sg0, sg1, so0, so1):
    wid = lax.axis_index("s") * NC + lax.axis_index("c")
    item0 = wid * IPW
    rows = (rows0, rows1)
    tb = (t0, t1)
    sg = (sg0, sg1)
    so = (so0, so1)

    def fire_gather(i, slot):
        pltpu.async_copy(
            table_hbm.at[idx_v.at[pl.ds(i * BT, BT)]], rows[slot], sg[slot]
        )

    def wait_gather(slot):
        pltpu.make_async_copy(
            table_hbm.at[pl.ds(0, BT)], rows[slot], sg[slot]
        ).wait()

    def fire_out(i, slot):
        it = item0 + i
        s = it // NBT
        bt = it % NBT
        for dt in range(4):
            pltpu.async_copy(
                tb[slot].at[dt],
                out_hbm.at[s, pl.ds(dt * (BT * 8 * BT) + bt * (8 * BT),
                                    8 * BT)],
                so[slot],
            )

    def wait_out(slot):
        for _ in range(4):
            pltpu.make_async_copy(
                tb[slot].at[0], out_hbm.at[0, pl.ds(0, 8 * BT)], so[slot]
            ).wait()

    def transpose(slot):
        lanes = lax.iota(jnp.int32, 16)
        for b0 in range(0, BT, 16):
            bvec = lanes + b0
            for d in range(D):
                col = plsc.load_gather(
                    rows[slot], [bvec, jnp.full((16,), d, jnp.int32)]
                )
                tb[slot][d // 8, pl.ds((d % 8) * BT + b0, 16)] = col

    pltpu.sync_copy(x_hbm.at[pl.ds(item0 * BT, IPW * BT)], idx_v)
    fire_gather(0, 0)
    fire_gather(1, 1)

    def body(p, carry):
        i = p * 2
        for sl in range(2):
            wait_gather(sl)

            @pl.when(p > 0)
            def _():
                wait_out(sl)

            transpose(sl)

            @pl.when(p < NPAIR - 1)
            def _():
                fire_gather(i + sl + 2, sl)

            fire_out(i + sl, sl)
        return carry

    lax.fori_loop(0, NPAIR, body, 0)
    wait_out(0)
    wait_out(1)


def kernel(x, table):
    xt_lin = jnp.transpose(x).reshape(B)
    out2 = _gather_kernel(xt_lin, table)
    out5 = out2.reshape(SEQ, 4, NBT, 8, BT)
    return jnp.transpose(out5, (2, 4, 0, 1, 3)).reshape(16384, SEQ, D)


# speedup vs baseline: 8.9771x; 1.8488x over previous
import functools

import jax
import jax.numpy as jnp
from jax import lax
from jax.experimental import pallas as pl
from jax.experimental.pallas import tpu as pltpu
from jax.experimental.pallas import tpu_sc as plsc

D = 32
SEQ = 50
BT = 128
NBT = 16384 // BT
B = 16384 * SEQ

NC, NS = 2, 16
NW = NC * NS
NITEM = SEQ * NBT
IPW = NITEM // NW
NPAIR = IPW // 2

OUTROW = 4 * BT * 8 * BT

_mesh = plsc.VectorSubcoreMesh(core_axis_name="c", subcore_axis_name="s")


@functools.partial(
    pl.kernel,
    out_type=jax.ShapeDtypeStruct((SEQ, OUTROW), jnp.float32),
    mesh=_mesh,
    compiler_params=pltpu.CompilerParams(
        use_tc_tiling_on_sc=False, needs_layout_passes=False
    ),
    scratch_types=[
        pltpu.VMEM((IPW * BT,), jnp.int32),
        pltpu.VMEM((BT, D), jnp.float32),
        pltpu.VMEM((BT, D), jnp.float32),
        pltpu.VMEM((4, 8 * BT), jnp.float32),
        pltpu.VMEM((4, 8 * BT), jnp.float32),
        pltpu.SemaphoreType.DMA,
        pltpu.SemaphoreType.DMA,
        pltpu.SemaphoreType.DMA,
        pltpu.SemaphoreType.DMA,
    ],
)
def _gather_kernel(x_hbm, table_hbm, out_hbm, idx_v, rows0, rows1, t0, t1,
                   sg0, sg1, so0, so1):
    wid = lax.axis_index("s") * NC + lax.axis_index("c")
    item0 = wid * IPW
    rows = (rows0, rows1)
    tb = (t0, t1)
    sg = (sg0, sg1)
    so = (so0, so1)

    def fire_gather(i, slot):
        pltpu.async_copy(
            table_hbm.at[idx_v.at[pl.ds(i * BT, BT)]], rows[slot], sg[slot]
        )

    def wait_gather(slot):
        pltpu.make_async_copy(
            table_hbm.at[pl.ds(0, BT)], rows[slot], sg[slot]
        ).wait()

    def fire_out(i, slot):
        it = item0 + i
        s = it // NBT
        bt = it % NBT
        for dt in range(4):
            pltpu.async_copy(
                tb[slot].at[dt],
                out_hbm.at[s, pl.ds(dt * (BT * 8 * BT) + bt * (8 * BT),
                                    8 * BT)],
                so[slot],
            )

    def wait_out(slot):
        for _ in range(4):
            pltpu.make_async_copy(
                tb[slot].at[0], out_hbm.at[0, pl.ds(0, 8 * BT)], so[slot]
            ).wait()

    lanes = lax.iota(jnp.int32, 16)
    bvecs = [lanes + b0 for b0 in range(0, BT, 16)]

    def transpose(slot):
        @plsc.parallel_loop(0, D, unroll=8)
        def _(d):
            dt = d // 8
            dr = (d % 8) * BT
            dv = jnp.full((16,), 0, jnp.int32) + d
            for j in range(BT // 16):
                col = plsc.load_gather(rows[slot], [bvecs[j], dv])
                tb[slot][dt, pl.ds(dr + j * 16, 16)] = col

    pltpu.sync_copy(x_hbm.at[pl.ds(item0 * BT, IPW * BT)], idx_v)
    fire_gather(0, 0)
    fire_gather(1, 1)

    def body(p, carry):
        i = p * 2
        for sl in range(2):
            wait_gather(sl)

            @pl.when(p > 0)
            def _():
                wait_out(sl)

            transpose(sl)

            @pl.when(p < NPAIR - 1)
            def _():
                fire_gather(i + sl + 2, sl)

            fire_out(i + sl, sl)
        return carry

    lax.fori_loop(0, NPAIR, body, 0)
    wait_out(0)
    wait_out(1)


def kernel(x, table):
    xt_lin = jnp.transpose(x).reshape(B)
    out2 = _gather_kernel(xt_lin, table)
    out5 = out2.reshape(SEQ, 4, NBT, 8, BT)
    return jnp.transpose(out5, (2, 4, 0, 1, 3)).reshape(16384, SEQ, D)


# speedup vs baseline: 19.6476x; 2.1886x over previous
import functools

import jax
import jax.numpy as jnp
from jax import lax
from jax.experimental import pallas as pl
from jax.experimental.pallas import tpu as pltpu
from jax.experimental.pallas import tpu_sc as plsc

D = 32
SEQ = 50
BT = 128
NBT = 16384 // BT
B = 16384 * SEQ
TP = BT + 1

NC, NS = 2, 16
NW = NC * NS
NITEM = SEQ * NBT
IPW = NITEM // NW
NPAIR = IPW // 2

_mesh = plsc.VectorSubcoreMesh(core_axis_name="c", subcore_axis_name="s")


@functools.partial(
    pl.kernel,
    out_type=jax.ShapeDtypeStruct((SEQ, 4, NBT, 8, BT), jnp.float32),
    mesh=_mesh,
    compiler_params=pltpu.CompilerParams(
        use_tc_tiling_on_sc=False, needs_layout_passes=False
    ),
    scratch_types=[
        pltpu.VMEM((IPW * BT,), jnp.int32),
        pltpu.VMEM((BT, D), jnp.float32),
        pltpu.VMEM((BT, D), jnp.float32),
        pltpu.VMEM((D, TP), jnp.float32),
        pltpu.VMEM((D, TP), jnp.float32),
        pltpu.SemaphoreType.DMA,
        pltpu.SemaphoreType.DMA,
        pltpu.SemaphoreType.DMA,
        pltpu.SemaphoreType.DMA,
    ],
)
def _gather_kernel(x_hbm, table_hbm, out_hbm, idx_v, rows0, rows1, t0, t1,
                   sg0, sg1, so0, so1):
    wid = lax.axis_index("s") * NC + lax.axis_index("c")
    item0 = wid * IPW
    rows = (rows0, rows1)
    tb = (t0, t1)
    sg = (sg0, sg1)
    so = (so0, so1)

    def fire_gather(i, slot):
        pltpu.async_copy(
            table_hbm.at[idx_v.at[pl.ds(i * BT, BT)]], rows[slot], sg[slot]
        )

    def wait_gather(slot):
        pltpu.make_async_copy(
            table_hbm.at[pl.ds(0, BT)], rows[slot], sg[slot]
        ).wait()

    def fire_out(i, slot):
        it = item0 + i
        s = it // NBT
        bt = it % NBT
        for dt in range(4):
            pltpu.async_copy(
                tb[slot].at[pl.ds(dt * 8, 8), pl.ds(0, BT)],
                out_hbm.at[s, dt, bt],
                so[slot],
            )

    def wait_out(slot):
        for _ in range(4):
            pltpu.make_async_copy(
                tb[slot].at[pl.ds(0, 8), pl.ds(0, BT)],
                out_hbm.at[0, 0, 0],
                so[slot],
            ).wait()

    lanes = lax.iota(jnp.int32, 16)
    lanes16 = lanes + 16

    def transpose(slot):
        @plsc.parallel_loop(0, BT, unroll=16)
        def _(b):
            bv = jnp.full((16,), 0, jnp.int32) + b
            v0 = rows[slot][b, pl.ds(0, 16)]
            v1 = rows[slot][b, pl.ds(16, 16)]
            plsc.store_scatter(tb[slot], [lanes, bv], v0)
            plsc.store_scatter(tb[slot], [lanes16, bv], v1)

    pltpu.sync_copy(x_hbm.at[pl.ds(item0 * BT, IPW * BT)], idx_v)
    fire_gather(0, 0)
    fire_gather(1, 1)

    def body(p, carry):
        i = p * 2
        for sl in range(2):
            wait_gather(sl)

            @pl.when(p > 0)
            def _():
                wait_out(sl)

            transpose(sl)

            @pl.when(p < NPAIR - 1)
            def _():
                fire_gather(i + sl + 2, sl)

            fire_out(i + sl, sl)
        return carry

    lax.fori_loop(0, NPAIR, body, 0)
    wait_out(0)
    wait_out(1)


def kernel(x, table):
    xt_lin = jnp.transpose(x).reshape(B)
    out5 = _gather_kernel(xt_lin, table)
    return jnp.transpose(out5, (2, 4, 0, 1, 3)).reshape(16384, SEQ, D)
